# Initial kernel scaffold; baseline (speedup 1.0000x reference)
import jax
import jax.numpy as jnp
from jax.experimental import pallas as pl


def kernel(x, edge_index, batch, W1, b1, W2, b2, Wf1, bf1, Wf2, bf2, Wo, bo):
    raise NotImplementedError("write your pallas kernel here")


# speedup vs baseline: 16.9674x; 16.9674x over previous
import functools

import jax
import jax.numpy as jnp
from jax import lax
from jax.experimental import pallas as pl
from jax.experimental.pallas import tpu as pltpu
from jax.experimental.pallas import tpu_sc as plsc

_N = 10000
_E = 320000
_D = 128
_G = 32
_CH = 80
_CPT = 125
_EPT = _E // 32
_RPS = _N // 16

def _mesh():
    return plsc.VectorSubcoreMesh(core_axis_name="c", subcore_axis_name="s",
                                  num_cores=2, num_subcores=16)


def _leaky(v):
    return jnp.where(v >= 0, v, 0.01 * v)


def _deg_body(dst_hbm, out_hbm, dstv, part):
    wid = lax.axis_index("c") * 16 + lax.axis_index("s")
    zero16 = jnp.zeros((16,), jnp.float32)
    ones16 = jnp.ones((16,), jnp.float32)

    def zrow(k, _):
        part[pl.ds(k * 16, 16)] = zero16
        return 0
    lax.fori_loop(0, _N // 16, zrow, 0)

    pltpu.sync_copy(dst_hbm.at[pl.ds(wid * _EPT, _EPT)], dstv)

    def step(k, _):
        idx = dstv[pl.ds(k * 16, 16)]
        plsc.addupdate_scatter(part, [idx], ones16)
        return 0
    lax.fori_loop(0, _EPT // 16, step, 0)

    pltpu.sync_copy(part, out_hbm.at[wid])


def _deg_call(f):
    return pl.kernel(
        f,
        mesh=_mesh(),
        compiler_params=pltpu.CompilerParams(needs_layout_passes=False),
        out_type=jax.ShapeDtypeStruct((32, _N), jnp.float32),
        scratch_types=[
            pltpu.VMEM((_EPT,), jnp.int32),
            pltpu.VMEM((_N,), jnp.float32),
        ],
    )


def _spmm_body(y_hbm, src_hbm, dst_hbm, out_hbm, srcv, dstv, rowsv, zbuf, acc, sem):
    cid = lax.axis_index("c")
    sid = lax.axis_index("s")
    wid = cid * 16 + sid
    zero16 = jnp.zeros((16,), jnp.float32)

    def zrow(r, _):
        for j in range(_D // 16):
            zbuf[r, pl.ds(j * 16, 16)] = zero16
        return 0
    lax.fori_loop(0, _CPT, zrow, 0)

    for j in range(_RPS // _CPT):
        pltpu.sync_copy(zbuf, acc.at[pl.ds(sid * _RPS + j * _CPT, _CPT)])

    pltpu.sync_copy(src_hbm.at[pl.ds(wid * _CPT, _CPT)], srcv)
    pltpu.sync_copy(dst_hbm.at[pl.ds(wid * _CPT, _CPT)], dstv)
    plsc.subcore_barrier()

    def chunk(i, _):
        pltpu.async_copy(y_hbm.at[srcv.at[i]], rowsv, sem).wait()
        pltpu.sync_copy(rowsv, acc.at[dstv.at[i]], add=True)
        return 0
    lax.fori_loop(0, _CPT, chunk, 0)

    plsc.subcore_barrier()
    pltpu.sync_copy(acc.at[pl.ds(sid * _RPS, _RPS)],
                    out_hbm.at[cid, pl.ds(sid * _RPS, _RPS)])


def _spmm_call(f):
    return pl.kernel(
        f,
        mesh=_mesh(),
        compiler_params=pltpu.CompilerParams(needs_layout_passes=False,
                                             use_tc_tiling_on_sc=False),
        out_type=jax.ShapeDtypeStruct((2, _N, _D), jnp.float32),
        scratch_types=[
            pltpu.VMEM((_CPT, _CH), jnp.int32),
            pltpu.VMEM((_CPT, _CH), jnp.int32),
            pltpu.VMEM((_CH, _D), jnp.float32),
            pltpu.VMEM((_CPT, _D), jnp.float32),
            pltpu.VMEM_SHARED((_N, _D), jnp.float32),
            pltpu.SemaphoreType.DMA,
        ],
    )


_BN = 1000


def _tc1_body(x_ref, w_ref, degp_ref, xw_ref, y_ref, dinv_ref):
    xw = jnp.dot(x_ref[...], w_ref[...], preferred_element_type=jnp.float32)
    deg = jnp.sum(degp_ref[...], axis=1) + 1.0
    dv = lax.rsqrt(deg)
    xw_ref[...] = xw
    y_ref[...] = xw * dv[:, None]
    dinv_ref[...] = dv[:, None]


def _tc1(x, W1, degp):
    return pl.pallas_call(
        _tc1_body,
        grid=(_N // _BN,),
        in_specs=[
            pl.BlockSpec((_BN, _D), lambda i: (i, 0)),
            pl.BlockSpec((_D, _D), lambda i: (0, 0)),
            pl.BlockSpec((_BN, 32), lambda i: (i, 0)),
        ],
        out_specs=[
            pl.BlockSpec((_BN, _D), lambda i: (i, 0)),
            pl.BlockSpec((_BN, _D), lambda i: (i, 0)),
            pl.BlockSpec((_BN, 1), lambda i: (i, 0)),
        ],
        out_shape=[
            jax.ShapeDtypeStruct((_N, _D), jnp.float32),
            jax.ShapeDtypeStruct((_N, _D), jnp.float32),
            jax.ShapeDtypeStruct((_N, 1), jnp.float32),
        ],
    )(x, W1, degp)


def _tc2_body(a0_ref, a1_ref, xw_ref, dinv_ref, w_ref, b_ref, xw2_ref, y2_ref):
    dv = dinv_ref[...]
    h = _leaky(dv * (a0_ref[...] + a1_ref[...]) + dv * dv * xw_ref[...] + b_ref[...])
    xw2 = jnp.dot(h, w_ref[...], preferred_element_type=jnp.float32)
    xw2_ref[...] = xw2
    y2_ref[...] = xw2 * dv


def _tc2(a0, a1, xw1, dinv, W2, b1):
    return pl.pallas_call(
        _tc2_body,
        grid=(_N // _BN,),
        in_specs=[
            pl.BlockSpec((_BN, _D), lambda i: (i, 0)),
            pl.BlockSpec((_BN, _D), lambda i: (i, 0)),
            pl.BlockSpec((_BN, _D), lambda i: (i, 0)),
            pl.BlockSpec((_BN, 1), lambda i: (i, 0)),
            pl.BlockSpec((_D, _D), lambda i: (0, 0)),
            pl.BlockSpec((1, _D), lambda i: (0, 0)),
        ],
        out_specs=[
            pl.BlockSpec((_BN, _D), lambda i: (i, 0)),
            pl.BlockSpec((_BN, _D), lambda i: (i, 0)),
        ],
        out_shape=[
            jax.ShapeDtypeStruct((_N, _D), jnp.float32),
            jax.ShapeDtypeStruct((_N, _D), jnp.float32),
        ],
    )(a0, a1, xw1, dinv, W2, b1)


def _tc3_body(a0_ref, a1_ref, xw_ref, dinv_ref, b_ref, batch_ref,
              wf1_ref, bf1_ref, wf2_ref, bf2_ref, wo_ref, bo_ref,
              out_ref, pooled, cnt):
    i = pl.program_id(0)

    @pl.when(i == 0)
    def _():
        pooled[...] = jnp.zeros_like(pooled)
        cnt[...] = jnp.zeros_like(cnt)

    dv = dinv_ref[...]
    h = _leaky(dv * (a0_ref[...] + a1_ref[...]) + dv * dv * xw_ref[...] + b_ref[...])
    b = batch_ref[0, 0, :]
    gi = lax.broadcasted_iota(jnp.int32, (_G, _BN), 0)
    oh = (b[None, :] == gi).astype(jnp.float32)
    pooled[...] += jnp.dot(oh, h, preferred_element_type=jnp.float32,
                           precision=lax.Precision.HIGHEST)
    cnt[...] += jnp.broadcast_to(jnp.sum(oh, axis=1, keepdims=True), (_G, _D))

    @pl.when(i == _N // _BN - 1)
    def _():
        g = pooled[...] / jnp.maximum(cnt[...], 1.0)
        f = _leaky(jnp.dot(g, wf1_ref[...], preferred_element_type=jnp.float32)
                   + bf1_ref[...])
        f = _leaky(jnp.dot(f, wf2_ref[...], preferred_element_type=jnp.float32)
                   + bf2_ref[...])
        out_ref[...] = jnp.dot(f, wo_ref[...], preferred_element_type=jnp.float32) \
            + bo_ref[...]


def _tc3(a0, a1, xw2, dinv, b2, batch3, Wf1, bf1, Wf2, bf2, Wo, bo):
    return pl.pallas_call(
        _tc3_body,
        grid=(_N // _BN,),
        in_specs=[
            pl.BlockSpec((_BN, _D), lambda i: (i, 0)),
            pl.BlockSpec((_BN, _D), lambda i: (i, 0)),
            pl.BlockSpec((_BN, _D), lambda i: (i, 0)),
            pl.BlockSpec((_BN, 1), lambda i: (i, 0)),
            pl.BlockSpec((1, _D), lambda i: (0, 0)),
            pl.BlockSpec((1, 1, _BN), lambda i: (i, 0, 0)),
            pl.BlockSpec((_D, 64), lambda i: (0, 0)),
            pl.BlockSpec((1, 64), lambda i: (0, 0)),
            pl.BlockSpec((64, 32), lambda i: (0, 0)),
            pl.BlockSpec((1, 32), lambda i: (0, 0)),
            pl.BlockSpec((32, 1), lambda i: (0, 0)),
            pl.BlockSpec((1, 1), lambda i: (0, 0)),
        ],
        out_specs=pl.BlockSpec((_G, 1), lambda i: (0, 0)),
        out_shape=jax.ShapeDtypeStruct((_G, 1), jnp.float32),
        scratch_shapes=[
            pltpu.VMEM((_G, _D), jnp.float32),
            pltpu.VMEM((_G, _D), jnp.float32),
        ],
    )(a0, a1, xw2, dinv, b2, batch3, Wf1, bf1, Wf2, bf2, Wo, bo)


def kernel(x, edge_index, batch, W1, b1, W2, b2, Wf1, bf1, Wf2, bf2, Wo, bo):
    src = edge_index[0]
    dst = edge_index[1]
    src2 = src.reshape(32 * _CPT, _CH)
    dst2 = dst.reshape(32 * _CPT, _CH)

    degp = _deg_call(_deg_body)(dst)
    xw1, y1, dinv = _tc1(x, W1, degp.T)
    accs1 = _spmm_call(_spmm_body)(y1, src2, dst2)
    xw2, y2 = _tc2(accs1[0], accs1[1], xw1, dinv, W2, b1.reshape(1, _D))
    accs2 = _spmm_call(_spmm_body)(y2, src2, dst2)
    out = _tc3(accs2[0], accs2[1], xw2, dinv, b2.reshape(1, _D),
               batch.reshape(_N // _BN, 1, _BN),
               Wf1, bf1.reshape(1, 64), Wf2, bf2.reshape(1, 32),
               Wo, bo.reshape(1, 1))
    return out


# speedup vs baseline: 25.7873x; 1.5198x over previous
import functools

import jax
import jax.numpy as jnp
from jax import lax
from jax.experimental import pallas as pl
from jax.experimental.pallas import tpu as pltpu
from jax.experimental.pallas import tpu_sc as plsc

_N = 10000
_E = 320000
_D = 128
_G = 32
_CH = 80
_CPT = 125
_EPT = _E // 32
_RPS = _N // 16
_ZR = 25

def _mesh():
    return plsc.VectorSubcoreMesh(core_axis_name="c", subcore_axis_name="s",
                                  num_cores=2, num_subcores=16)


def _leaky(v):
    return jnp.where(v >= 0, v, 0.01 * v)


def _deg_body(dst_hbm, out_hbm, dstv, part):
    wid = lax.axis_index("c") * 16 + lax.axis_index("s")
    zero16 = jnp.zeros((16,), jnp.float32)
    ones16 = jnp.ones((16,), jnp.float32)

    def zrow(k, _):
        part[pl.ds(k * 16, 16)] = zero16
        return 0
    lax.fori_loop(0, _N // 16, zrow, 0)

    pltpu.sync_copy(dst_hbm.at[pl.ds(wid * _EPT, _EPT)], dstv)

    def step(k, _):
        idx = dstv[pl.ds(k * 16, 16)]
        plsc.addupdate_scatter(part, [idx], ones16)
        return 0
    lax.fori_loop(0, _EPT // 16, step, 0)

    pltpu.sync_copy(part, out_hbm.at[wid])


def _deg_call(f):
    return pl.kernel(
        f,
        mesh=_mesh(),
        compiler_params=pltpu.CompilerParams(needs_layout_passes=False),
        out_type=jax.ShapeDtypeStruct((32, _N), jnp.float32),
        scratch_types=[
            pltpu.VMEM((_EPT,), jnp.int32),
            pltpu.VMEM((_N,), jnp.float32),
        ],
    )


def _spmm_body(y_hbm, src_hbm, dst_hbm, out_hbm, srcv, dstv, rowsv0, rowsv1,
               zbuf, acc, sem0, sem1):
    cid = lax.axis_index("c")
    sid = lax.axis_index("s")
    wid = cid * 16 + sid
    zero16 = jnp.zeros((16,), jnp.float32)

    def zrow(r, _):
        for j in range(_D // 16):
            zbuf[r, pl.ds(j * 16, 16)] = zero16
        return 0
    lax.fori_loop(0, _ZR, zrow, 0)

    def zslab(j, _):
        pltpu.sync_copy(zbuf, acc.at[pl.ds(sid * _RPS + j * _ZR, _ZR)])
        return 0
    lax.fori_loop(0, _RPS // _ZR, zslab, 0)

    pltpu.sync_copy(src_hbm.at[pl.ds(wid * _CPT, _CPT)], srcv)
    pltpu.sync_copy(dst_hbm.at[pl.ds(wid * _CPT, _CPT)], dstv)
    plsc.subcore_barrier()

    bufs = (rowsv0, rowsv1)
    sems = (sem0, sem1)
    pltpu.async_copy(y_hbm.at[srcv.at[0]], rowsv0, sem0)
    pltpu.async_copy(y_hbm.at[srcv.at[1]], rowsv1, sem1)

    def pair(j, _):
        k = j * 2
        for b in range(2):
            pltpu.make_async_copy(y_hbm.at[srcv.at[k + b]], bufs[b],
                                  sems[b]).wait()
            pltpu.sync_copy(bufs[b], acc.at[dstv.at[k + b]], add=True)

            @pl.when(k + b + 2 < _CPT)
            def _():
                pltpu.async_copy(y_hbm.at[srcv.at[k + b + 2]], bufs[b], sems[b])
        return 0
    lax.fori_loop(0, _CPT // 2, pair, 0)

    pltpu.make_async_copy(y_hbm.at[srcv.at[_CPT - 1]], rowsv0, sem0).wait()
    pltpu.sync_copy(rowsv0, acc.at[dstv.at[_CPT - 1]], add=True)

    plsc.subcore_barrier()
    pltpu.sync_copy(acc.at[pl.ds(sid * _RPS, _RPS)],
                    out_hbm.at[cid, pl.ds(sid * _RPS, _RPS)])


def _spmm_call(f):
    return pl.kernel(
        f,
        mesh=_mesh(),
        compiler_params=pltpu.CompilerParams(needs_layout_passes=False,
                                             use_tc_tiling_on_sc=False),
        out_type=jax.ShapeDtypeStruct((2, _N, _D), jnp.float32),
        scratch_types=[
            pltpu.VMEM((_CPT, _CH), jnp.int32),
            pltpu.VMEM((_CPT, _CH), jnp.int32),
            pltpu.VMEM((_CH, _D), jnp.float32),
            pltpu.VMEM((_CH, _D), jnp.float32),
            pltpu.VMEM((_ZR, _D), jnp.float32),
            pltpu.VMEM_SHARED((_N, _D), jnp.float32),
            pltpu.SemaphoreType.DMA,
            pltpu.SemaphoreType.DMA,
        ],
    )


_BN = 1000


def _tc1_body(x_ref, w_ref, degp_ref, xw_ref, y_ref, dinv_ref):
    xw = jnp.dot(x_ref[...], w_ref[...], preferred_element_type=jnp.float32)
    deg = jnp.sum(degp_ref[...], axis=1) + 1.0
    dv = lax.rsqrt(deg)
    xw_ref[...] = xw
    y_ref[...] = xw * dv[:, None]
    dinv_ref[...] = dv[:, None]


def _tc1(x, W1, degp):
    return pl.pallas_call(
        _tc1_body,
        grid=(_N // _BN,),
        in_specs=[
            pl.BlockSpec((_BN, _D), lambda i: (i, 0)),
            pl.BlockSpec((_D, _D), lambda i: (0, 0)),
            pl.BlockSpec((_BN, 32), lambda i: (i, 0)),
        ],
        out_specs=[
            pl.BlockSpec((_BN, _D), lambda i: (i, 0)),
            pl.BlockSpec((_BN, _D), lambda i: (i, 0)),
            pl.BlockSpec((_BN, 1), lambda i: (i, 0)),
        ],
        out_shape=[
            jax.ShapeDtypeStruct((_N, _D), jnp.float32),
            jax.ShapeDtypeStruct((_N, _D), jnp.float32),
            jax.ShapeDtypeStruct((_N, 1), jnp.float32),
        ],
    )(x, W1, degp)


def _tc2_body(a0_ref, a1_ref, xw_ref, dinv_ref, w_ref, b_ref, xw2_ref, y2_ref):
    dv = dinv_ref[...]
    h = _leaky(dv * (a0_ref[...] + a1_ref[...]) + dv * dv * xw_ref[...] + b_ref[...])
    xw2 = jnp.dot(h, w_ref[...], preferred_element_type=jnp.float32)
    xw2_ref[...] = xw2
    y2_ref[...] = xw2 * dv


def _tc2(a0, a1, xw1, dinv, W2, b1):
    return pl.pallas_call(
        _tc2_body,
        grid=(_N // _BN,),
        in_specs=[
            pl.BlockSpec((_BN, _D), lambda i: (i, 0)),
            pl.BlockSpec((_BN, _D), lambda i: (i, 0)),
            pl.BlockSpec((_BN, _D), lambda i: (i, 0)),
            pl.BlockSpec((_BN, 1), lambda i: (i, 0)),
            pl.BlockSpec((_D, _D), lambda i: (0, 0)),
            pl.BlockSpec((1, _D), lambda i: (0, 0)),
        ],
        out_specs=[
            pl.BlockSpec((_BN, _D), lambda i: (i, 0)),
            pl.BlockSpec((_BN, _D), lambda i: (i, 0)),
        ],
        out_shape=[
            jax.ShapeDtypeStruct((_N, _D), jnp.float32),
            jax.ShapeDtypeStruct((_N, _D), jnp.float32),
        ],
    )(a0, a1, xw1, dinv, W2, b1)


def _tc3_body(a0_ref, a1_ref, xw_ref, dinv_ref, b_ref, batch_ref,
              wf1_ref, bf1_ref, wf2_ref, bf2_ref, wo_ref, bo_ref,
              out_ref, pooled, cnt):
    i = pl.program_id(0)

    @pl.when(i == 0)
    def _():
        pooled[...] = jnp.zeros_like(pooled)
        cnt[...] = jnp.zeros_like(cnt)

    dv = dinv_ref[...]
    h = _leaky(dv * (a0_ref[...] + a1_ref[...]) + dv * dv * xw_ref[...] + b_ref[...])
    b = batch_ref[0, 0, :]
    gi = lax.broadcasted_iota(jnp.int32, (_G, _BN), 0)
    oh = (b[None, :] == gi).astype(jnp.float32)
    pooled[...] += jnp.dot(oh, h, preferred_element_type=jnp.float32,
                           precision=lax.Precision.HIGHEST)
    cnt[...] += jnp.broadcast_to(jnp.sum(oh, axis=1, keepdims=True), (_G, _D))

    @pl.when(i == _N // _BN - 1)
    def _():
        g = pooled[...] / jnp.maximum(cnt[...], 1.0)
        f = _leaky(jnp.dot(g, wf1_ref[...], preferred_element_type=jnp.float32)
                   + bf1_ref[...])
        f = _leaky(jnp.dot(f, wf2_ref[...], preferred_element_type=jnp.float32)
                   + bf2_ref[...])
        out_ref[...] = jnp.dot(f, wo_ref[...], preferred_element_type=jnp.float32) \
            + bo_ref[...]


def _tc3(a0, a1, xw2, dinv, b2, batch3, Wf1, bf1, Wf2, bf2, Wo, bo):
    return pl.pallas_call(
        _tc3_body,
        grid=(_N // _BN,),
        in_specs=[
            pl.BlockSpec((_BN, _D), lambda i: (i, 0)),
            pl.BlockSpec((_BN, _D), lambda i: (i, 0)),
            pl.BlockSpec((_BN, _D), lambda i: (i, 0)),
            pl.BlockSpec((_BN, 1), lambda i: (i, 0)),
            pl.BlockSpec((1, _D), lambda i: (0, 0)),
            pl.BlockSpec((1, 1, _BN), lambda i: (i, 0, 0)),
            pl.BlockSpec((_D, 64), lambda i: (0, 0)),
            pl.BlockSpec((1, 64), lambda i: (0, 0)),
            pl.BlockSpec((64, 32), lambda i: (0, 0)),
            pl.BlockSpec((1, 32), lambda i: (0, 0)),
            pl.BlockSpec((32, 1), lambda i: (0, 0)),
            pl.BlockSpec((1, 1), lambda i: (0, 0)),
        ],
        out_specs=pl.BlockSpec((_G, 1), lambda i: (0, 0)),
        out_shape=jax.ShapeDtypeStruct((_G, 1), jnp.float32),
        scratch_shapes=[
            pltpu.VMEM((_G, _D), jnp.float32),
            pltpu.VMEM((_G, _D), jnp.float32),
        ],
    )(a0, a1, xw2, dinv, b2, batch3, Wf1, bf1, Wf2, bf2, Wo, bo)


def kernel(x, edge_index, batch, W1, b1, W2, b2, Wf1, bf1, Wf2, bf2, Wo, bo):
    src = edge_index[0]
    dst = edge_index[1]
    src2 = src.reshape(32 * _CPT, _CH)
    dst2 = dst.reshape(32 * _CPT, _CH)

    degp = _deg_call(_deg_body)(dst)
    xw1, y1, dinv = _tc1(x, W1, degp.T)
    accs1 = _spmm_call(_spmm_body)(y1, src2, dst2)
    xw2, y2 = _tc2(accs1[0], accs1[1], xw1, dinv, W2, b1.reshape(1, _D))
    accs2 = _spmm_call(_spmm_body)(y2, src2, dst2)
    out = _tc3(accs2[0], accs2[1], xw2, dinv, b2.reshape(1, _D),
               batch.reshape(_N // _BN, 1, _BN),
               Wf1, bf1.reshape(1, 64), Wf2, bf2.reshape(1, 32),
               Wo, bo.reshape(1, 1))
    return out


# speedup vs baseline: 27.4673x; 1.0651x over previous
import functools

import jax
import jax.numpy as jnp
from jax import lax
from jax.experimental import pallas as pl
from jax.experimental.pallas import tpu as pltpu
from jax.experimental.pallas import tpu_sc as plsc

_N = 10000
_E = 320000
_D = 128
_G = 32
_CH = 80
_CPT = 125
_EPT = _E // 32
_RPS = _N // 16
_ZR = 25

def _mesh():
    return plsc.VectorSubcoreMesh(core_axis_name="c", subcore_axis_name="s",
                                  num_cores=2, num_subcores=16)


def _leaky(v):
    return jnp.where(v >= 0, v, 0.01 * v)


def _deg_body(dst_hbm, out_hbm, dstv, part):
    wid = lax.axis_index("c") * 16 + lax.axis_index("s")
    zero16 = jnp.zeros((16,), jnp.float32)
    ones16 = jnp.ones((16,), jnp.float32)

    def zrow(k, _):
        part[pl.ds(k * 16, 16)] = zero16
        return 0
    lax.fori_loop(0, _N // 16, zrow, 0)

    pltpu.sync_copy(dst_hbm.at[pl.ds(wid * _EPT, _EPT)], dstv)

    def step(k, _):
        idx = dstv[pl.ds(k * 16, 16)]
        plsc.addupdate_scatter(part, [idx], ones16)
        return 0
    lax.fori_loop(0, _EPT // 16, step, 0)

    pltpu.sync_copy(part, out_hbm.at[wid])


def _deg_call(f):
    return pl.kernel(
        f,
        mesh=_mesh(),
        compiler_params=pltpu.CompilerParams(needs_layout_passes=False),
        out_type=jax.ShapeDtypeStruct((32, _N), jnp.float32),
        scratch_types=[
            pltpu.VMEM((_EPT,), jnp.int32),
            pltpu.VMEM((_N,), jnp.float32),
        ],
    )


def _spmm_body(y_hbm, src_hbm, dst_hbm, out_hbm, srcv, dstv, rowsv0, rowsv1,
               zbuf, acc, sem0, sem1):
    cid = lax.axis_index("c")
    sid = lax.axis_index("s")
    wid = cid * 16 + sid
    zero16 = jnp.zeros((16,), jnp.float32)

    def zrow(r, _):
        for j in range(_D // 16):
            zbuf[r, pl.ds(j * 16, 16)] = zero16
        return 0
    lax.fori_loop(0, _ZR, zrow, 0)

    def zslab(j, _):
        pltpu.async_copy(zbuf, acc.at[pl.ds(sid * _RPS + j * _ZR, _ZR)], sem0)
        return 0
    lax.fori_loop(0, _RPS // _ZR, zslab, 0)

    pltpu.sync_copy(src_hbm.at[pl.ds(wid * _CPT, _CPT)], srcv)
    pltpu.sync_copy(dst_hbm.at[pl.ds(wid * _CPT, _CPT)], dstv)

    def zdrain(j, _):
        pltpu.make_async_copy(zbuf, acc.at[pl.ds(sid * _RPS + j * _ZR, _ZR)],
                              sem0).wait()
        return 0
    lax.fori_loop(0, _RPS // _ZR, zdrain, 0)
    plsc.subcore_barrier()

    bufs = (rowsv0, rowsv1)
    sems = (sem0, sem1)
    pltpu.async_copy(y_hbm.at[srcv.at[0]], rowsv0, sem0)
    pltpu.async_copy(y_hbm.at[srcv.at[1]], rowsv1, sem1)

    def pair(j, _):
        k = j * 2
        for b in range(2):
            pltpu.make_async_copy(y_hbm.at[srcv.at[k + b]], bufs[b],
                                  sems[b]).wait()
            pltpu.sync_copy(bufs[b], acc.at[dstv.at[k + b]], add=True)

            @pl.when(k + b + 2 < _CPT)
            def _():
                pltpu.async_copy(y_hbm.at[srcv.at[k + b + 2]], bufs[b], sems[b])
        return 0
    lax.fori_loop(0, _CPT // 2, pair, 0)

    pltpu.make_async_copy(y_hbm.at[srcv.at[_CPT - 1]], rowsv0, sem0).wait()
    pltpu.sync_copy(rowsv0, acc.at[dstv.at[_CPT - 1]], add=True)

    plsc.subcore_barrier()
    pltpu.sync_copy(acc.at[pl.ds(sid * _RPS, _RPS)],
                    out_hbm.at[cid, pl.ds(sid * _RPS, _RPS)])


def _spmm_call(f):
    return pl.kernel(
        f,
        mesh=_mesh(),
        compiler_params=pltpu.CompilerParams(needs_layout_passes=False,
                                             use_tc_tiling_on_sc=False),
        out_type=jax.ShapeDtypeStruct((2, _N, _D), jnp.float32),
        scratch_types=[
            pltpu.VMEM((_CPT, _CH), jnp.int32),
            pltpu.VMEM((_CPT, _CH), jnp.int32),
            pltpu.VMEM((_CH, _D), jnp.float32),
            pltpu.VMEM((_CH, _D), jnp.float32),
            pltpu.VMEM((_ZR, _D), jnp.float32),
            pltpu.VMEM_SHARED((_N, _D), jnp.float32),
            pltpu.SemaphoreType.DMA,
            pltpu.SemaphoreType.DMA,
        ],
    )


_BN = 1000


def _tc1_body(x_ref, w_ref, degp_ref, xw_ref, y_ref, dinv_ref):
    xw = jnp.dot(x_ref[...], w_ref[...], preferred_element_type=jnp.float32)
    deg = jnp.sum(degp_ref[...], axis=1) + 1.0
    dv = lax.rsqrt(deg)
    xw_ref[...] = xw
    y_ref[...] = xw * dv[:, None]
    dinv_ref[...] = dv[:, None]


def _tc1(x, W1, degp):
    return pl.pallas_call(
        _tc1_body,
        grid=(_N // _BN,),
        in_specs=[
            pl.BlockSpec((_BN, _D), lambda i: (i, 0)),
            pl.BlockSpec((_D, _D), lambda i: (0, 0)),
            pl.BlockSpec((_BN, 32), lambda i: (i, 0)),
        ],
        out_specs=[
            pl.BlockSpec((_BN, _D), lambda i: (i, 0)),
            pl.BlockSpec((_BN, _D), lambda i: (i, 0)),
            pl.BlockSpec((_BN, 1), lambda i: (i, 0)),
        ],
        out_shape=[
            jax.ShapeDtypeStruct((_N, _D), jnp.float32),
            jax.ShapeDtypeStruct((_N, _D), jnp.float32),
            jax.ShapeDtypeStruct((_N, 1), jnp.float32),
        ],
    )(x, W1, degp)


def _tc2_body(a0_ref, a1_ref, xw_ref, dinv_ref, w_ref, b_ref, xw2_ref, y2_ref):
    dv = dinv_ref[...]
    h = _leaky(dv * (a0_ref[0] + a1_ref[0]) + dv * dv * xw_ref[...] + b_ref[...])
    xw2 = jnp.dot(h, w_ref[...], preferred_element_type=jnp.float32)
    xw2_ref[...] = xw2
    y2_ref[...] = xw2 * dv


def _tc2(accs, xw1, dinv, W2, b1):
    return pl.pallas_call(
        _tc2_body,
        grid=(_N // _BN,),
        in_specs=[
            pl.BlockSpec((1, _BN, _D), lambda i: (0, i, 0)),
            pl.BlockSpec((1, _BN, _D), lambda i: (1, i, 0)),
            pl.BlockSpec((_BN, _D), lambda i: (i, 0)),
            pl.BlockSpec((_BN, 1), lambda i: (i, 0)),
            pl.BlockSpec((_D, _D), lambda i: (0, 0)),
            pl.BlockSpec((1, _D), lambda i: (0, 0)),
        ],
        out_specs=[
            pl.BlockSpec((_BN, _D), lambda i: (i, 0)),
            pl.BlockSpec((_BN, _D), lambda i: (i, 0)),
        ],
        out_shape=[
            jax.ShapeDtypeStruct((_N, _D), jnp.float32),
            jax.ShapeDtypeStruct((_N, _D), jnp.float32),
        ],
    )(accs, accs, xw1, dinv, W2, b1)


def _tc3_body(a0_ref, a1_ref, xw_ref, dinv_ref, b_ref, batch_ref,
              wf1_ref, bf1_ref, wf2_ref, bf2_ref, wo_ref, bo_ref,
              out_ref, pooled, cnt):
    i = pl.program_id(0)

    @pl.when(i == 0)
    def _():
        pooled[...] = jnp.zeros_like(pooled)
        cnt[...] = jnp.zeros_like(cnt)

    dv = dinv_ref[...]
    h = _leaky(dv * (a0_ref[0] + a1_ref[0]) + dv * dv * xw_ref[...] + b_ref[...])
    b = batch_ref[0, 0, :]
    gi = lax.broadcasted_iota(jnp.int32, (_G, _BN), 0)
    oh = (b[None, :] == gi).astype(jnp.float32)
    pooled[...] += jnp.dot(oh, h, preferred_element_type=jnp.float32,
                           precision=lax.Precision.HIGHEST)
    cnt[...] += jnp.broadcast_to(jnp.sum(oh, axis=1, keepdims=True), (_G, _D))

    @pl.when(i == _N // _BN - 1)
    def _():
        g = pooled[...] / jnp.maximum(cnt[...], 1.0)
        f = _leaky(jnp.dot(g, wf1_ref[...], preferred_element_type=jnp.float32)
                   + bf1_ref[...])
        f = _leaky(jnp.dot(f, wf2_ref[...], preferred_element_type=jnp.float32)
                   + bf2_ref[...])
        out_ref[...] = jnp.dot(f, wo_ref[...], preferred_element_type=jnp.float32) \
            + bo_ref[...]


def _tc3(accs, xw2, dinv, b2, batch3, Wf1, bf1, Wf2, bf2, Wo, bo):
    return pl.pallas_call(
        _tc3_body,
        grid=(_N // _BN,),
        in_specs=[
            pl.BlockSpec((1, _BN, _D), lambda i: (0, i, 0)),
            pl.BlockSpec((1, _BN, _D), lambda i: (1, i, 0)),
            pl.BlockSpec((_BN, _D), lambda i: (i, 0)),
            pl.BlockSpec((_BN, 1), lambda i: (i, 0)),
            pl.BlockSpec((1, _D), lambda i: (0, 0)),
            pl.BlockSpec((1, 1, _BN), lambda i: (i, 0, 0)),
            pl.BlockSpec((_D, 64), lambda i: (0, 0)),
            pl.BlockSpec((1, 64), lambda i: (0, 0)),
            pl.BlockSpec((64, 32), lambda i: (0, 0)),
            pl.BlockSpec((1, 32), lambda i: (0, 0)),
            pl.BlockSpec((32, 1), lambda i: (0, 0)),
            pl.BlockSpec((1, 1), lambda i: (0, 0)),
        ],
        out_specs=pl.BlockSpec((_G, 1), lambda i: (0, 0)),
        out_shape=jax.ShapeDtypeStruct((_G, 1), jnp.float32),
        scratch_shapes=[
            pltpu.VMEM((_G, _D), jnp.float32),
            pltpu.VMEM((_G, _D), jnp.float32),
        ],
    )(accs, accs, xw2, dinv, b2, batch3, Wf1, bf1, Wf2, bf2, Wo, bo)


def kernel(x, edge_index, batch, W1, b1, W2, b2, Wf1, bf1, Wf2, bf2, Wo, bo):
    src = edge_index[0]
    dst = edge_index[1]
    src2 = src.reshape(32 * _CPT, _CH)
    dst2 = dst.reshape(32 * _CPT, _CH)

    degp = _deg_call(_deg_body)(dst)
    xw1, y1, dinv = _tc1(x, W1, degp.T)
    accs1 = _spmm_call(_spmm_body)(y1, src2, dst2)
    xw2, y2 = _tc2(accs1, xw1, dinv, W2, b1.reshape(1, _D))
    accs2 = _spmm_call(_spmm_body)(y2, src2, dst2)
    out = _tc3(accs2, xw2, dinv, b2.reshape(1, _D),
               batch.reshape(_N // _BN, 1, _BN),
               Wf1, bf1.reshape(1, 64), Wf2, bf2.reshape(1, 32),
               Wo, bo.reshape(1, 1))
    return out


# speedup vs baseline: 27.5866x; 1.0043x over previous
import functools

import jax
import jax.numpy as jnp
from jax import lax
from jax.experimental import pallas as pl
from jax.experimental.pallas import tpu as pltpu
from jax.experimental.pallas import tpu_sc as plsc

_N = 10000
_E = 320000
_D = 128
_G = 32
_CH = 80
_CPT = 125
_EPT = _E // 32
_RPS = _N // 16
_ZR = 25

def _mesh():
    return plsc.VectorSubcoreMesh(core_axis_name="c", subcore_axis_name="s",
                                  num_cores=2, num_subcores=16)


def _leaky(v):
    return jnp.where(v >= 0, v, 0.01 * v)


def _deg_body(dst_hbm, out_hbm, dstv, part):
    wid = lax.axis_index("c") * 16 + lax.axis_index("s")
    zero16 = jnp.zeros((16,), jnp.float32)
    ones16 = jnp.ones((16,), jnp.float32)

    def zrow(k, _):
        part[pl.ds(k * 16, 16)] = zero16
        return 0
    lax.fori_loop(0, _N // 16, zrow, 0)

    pltpu.sync_copy(dst_hbm.at[pl.ds(wid * _EPT, _EPT)], dstv)

    def step(k, _):
        idx = dstv[pl.ds(k * 16, 16)]
        plsc.addupdate_scatter(part, [idx], ones16)
        return 0
    lax.fori_loop(0, _EPT // 16, step, 0)

    pltpu.sync_copy(part, out_hbm.at[wid])


def _deg_call(f):
    return pl.kernel(
        f,
        mesh=_mesh(),
        compiler_params=pltpu.CompilerParams(needs_layout_passes=False),
        out_type=jax.ShapeDtypeStruct((32, _N), jnp.float32),
        scratch_types=[
            pltpu.VMEM((_EPT,), jnp.int32),
            pltpu.VMEM((_N,), jnp.float32),
        ],
    )


def _spmm_body(y_hbm, src_hbm, dst_hbm, out_hbm, srcv, dstv, rowsv0, rowsv1,
               zbuf, acc, semz, semg0, semg1):
    cid = lax.axis_index("c")
    sid = lax.axis_index("s")
    wid = cid * 16 + sid
    zero16 = jnp.zeros((16,), jnp.float32)

    def zrow(r, _):
        for j in range(_D // 16):
            zbuf[r, pl.ds(j * 16, 16)] = zero16
        return 0
    lax.fori_loop(0, _ZR, zrow, 0)

    def zslab(j, _):
        pltpu.async_copy(zbuf, acc.at[pl.ds(sid * _RPS + j * _ZR, _ZR)], semz)
        return 0
    lax.fori_loop(0, _RPS // _ZR, zslab, 0)

    pltpu.sync_copy(src_hbm.at[pl.ds(wid * _CPT, _CPT)], srcv)
    pltpu.sync_copy(dst_hbm.at[pl.ds(wid * _CPT, _CPT)], dstv)

    bufs = (rowsv0, rowsv1)
    sems = (semg0, semg1)
    pltpu.async_copy(y_hbm.at[srcv.at[0]], rowsv0, semg0)
    pltpu.async_copy(y_hbm.at[srcv.at[1]], rowsv1, semg1)

    def zdrain(j, _):
        pltpu.make_async_copy(zbuf, acc.at[pl.ds(sid * _RPS + j * _ZR, _ZR)],
                              semz).wait()
        return 0
    lax.fori_loop(0, _RPS // _ZR, zdrain, 0)
    plsc.subcore_barrier()


    def pair(j, _):
        k = j * 2
        for b in range(2):
            pltpu.make_async_copy(y_hbm.at[srcv.at[k + b]], bufs[b],
                                  sems[b]).wait()
            pltpu.sync_copy(bufs[b], acc.at[dstv.at[k + b]], add=True)

            @pl.when(k + b + 2 < _CPT)
            def _():
                pltpu.async_copy(y_hbm.at[srcv.at[k + b + 2]], bufs[b], sems[b])
        return 0
    lax.fori_loop(0, _CPT // 2, pair, 0)

    pltpu.make_async_copy(y_hbm.at[srcv.at[_CPT - 1]], rowsv0, semg0).wait()
    pltpu.sync_copy(rowsv0, acc.at[dstv.at[_CPT - 1]], add=True)

    plsc.subcore_barrier()
    pltpu.sync_copy(acc.at[pl.ds(sid * _RPS, _RPS)],
                    out_hbm.at[cid, pl.ds(sid * _RPS, _RPS)])


def _spmm_call(f):
    return pl.kernel(
        f,
        mesh=_mesh(),
        compiler_params=pltpu.CompilerParams(needs_layout_passes=False,
                                             use_tc_tiling_on_sc=False),
        out_type=jax.ShapeDtypeStruct((2, _N, _D), jnp.float32),
        scratch_types=[
            pltpu.VMEM((_CPT, _CH), jnp.int32),
            pltpu.VMEM((_CPT, _CH), jnp.int32),
            pltpu.VMEM((_CH, _D), jnp.float32),
            pltpu.VMEM((_CH, _D), jnp.float32),
            pltpu.VMEM((_ZR, _D), jnp.float32),
            pltpu.VMEM_SHARED((_N, _D), jnp.float32),
            pltpu.SemaphoreType.DMA,
            pltpu.SemaphoreType.DMA,
            pltpu.SemaphoreType.DMA,
        ],
    )


_BN = 1000


def _tc1_body(x_ref, w_ref, degp_ref, xw_ref, y_ref, dinv_ref):
    xw = jnp.dot(x_ref[...], w_ref[...], preferred_element_type=jnp.float32)
    deg = jnp.sum(degp_ref[...], axis=1) + 1.0
    dv = lax.rsqrt(deg)
    xw_ref[...] = xw
    y_ref[...] = xw * dv[:, None]
    dinv_ref[...] = dv[:, None]


def _tc1(x, W1, degp):
    return pl.pallas_call(
        _tc1_body,
        grid=(_N // _BN,),
        in_specs=[
            pl.BlockSpec((_BN, _D), lambda i: (i, 0)),
            pl.BlockSpec((_D, _D), lambda i: (0, 0)),
            pl.BlockSpec((_BN, 32), lambda i: (i, 0)),
        ],
        out_specs=[
            pl.BlockSpec((_BN, _D), lambda i: (i, 0)),
            pl.BlockSpec((_BN, _D), lambda i: (i, 0)),
            pl.BlockSpec((_BN, 1), lambda i: (i, 0)),
        ],
        out_shape=[
            jax.ShapeDtypeStruct((_N, _D), jnp.float32),
            jax.ShapeDtypeStruct((_N, _D), jnp.float32),
            jax.ShapeDtypeStruct((_N, 1), jnp.float32),
        ],
    )(x, W1, degp)


def _tc2_body(a0_ref, a1_ref, xw_ref, dinv_ref, w_ref, b_ref, xw2_ref, y2_ref):
    dv = dinv_ref[...]
    h = _leaky(dv * (a0_ref[0] + a1_ref[0]) + dv * dv * xw_ref[...] + b_ref[...])
    xw2 = jnp.dot(h, w_ref[...], preferred_element_type=jnp.float32)
    xw2_ref[...] = xw2
    y2_ref[...] = xw2 * dv


def _tc2(accs, xw1, dinv, W2, b1):
    return pl.pallas_call(
        _tc2_body,
        grid=(_N // _BN,),
        in_specs=[
            pl.BlockSpec((1, _BN, _D), lambda i: (0, i, 0)),
            pl.BlockSpec((1, _BN, _D), lambda i: (1, i, 0)),
            pl.BlockSpec((_BN, _D), lambda i: (i, 0)),
            pl.BlockSpec((_BN, 1), lambda i: (i, 0)),
            pl.BlockSpec((_D, _D), lambda i: (0, 0)),
            pl.BlockSpec((1, _D), lambda i: (0, 0)),
        ],
        out_specs=[
            pl.BlockSpec((_BN, _D), lambda i: (i, 0)),
            pl.BlockSpec((_BN, _D), lambda i: (i, 0)),
        ],
        out_shape=[
            jax.ShapeDtypeStruct((_N, _D), jnp.float32),
            jax.ShapeDtypeStruct((_N, _D), jnp.float32),
        ],
    )(accs, accs, xw1, dinv, W2, b1)


def _tc3_body(a0_ref, a1_ref, xw_ref, dinv_ref, b_ref, batch_ref,
              wf1_ref, bf1_ref, wf2_ref, bf2_ref, wo_ref, bo_ref,
              out_ref, pooled, cnt):
    i = pl.program_id(0)

    @pl.when(i == 0)
    def _():
        pooled[...] = jnp.zeros_like(pooled)
        cnt[...] = jnp.zeros_like(cnt)

    dv = dinv_ref[...]
    h = _leaky(dv * (a0_ref[0] + a1_ref[0]) + dv * dv * xw_ref[...] + b_ref[...])
    b = batch_ref[0, 0, :]
    gi = lax.broadcasted_iota(jnp.int32, (_G, _BN), 0)
    oh = (b[None, :] == gi).astype(jnp.float32)
    pooled[...] += jnp.dot(oh, h, preferred_element_type=jnp.float32,
                           precision=lax.Precision.HIGHEST)
    cnt[...] += jnp.broadcast_to(jnp.sum(oh, axis=1, keepdims=True), (_G, _D))

    @pl.when(i == _N // _BN - 1)
    def _():
        g = pooled[...] / jnp.maximum(cnt[...], 1.0)
        f = _leaky(jnp.dot(g, wf1_ref[...], preferred_element_type=jnp.float32)
                   + bf1_ref[...])
        f = _leaky(jnp.dot(f, wf2_ref[...], preferred_element_type=jnp.float32)
                   + bf2_ref[...])
        out_ref[...] = jnp.dot(f, wo_ref[...], preferred_element_type=jnp.float32) \
            + bo_ref[...]


def _tc3(accs, xw2, dinv, b2, batch3, Wf1, bf1, Wf2, bf2, Wo, bo):
    return pl.pallas_call(
        _tc3_body,
        grid=(_N // _BN,),
        in_specs=[
            pl.BlockSpec((1, _BN, _D), lambda i: (0, i, 0)),
            pl.BlockSpec((1, _BN, _D), lambda i: (1, i, 0)),
            pl.BlockSpec((_BN, _D), lambda i: (i, 0)),
            pl.BlockSpec((_BN, 1), lambda i: (i, 0)),
            pl.BlockSpec((1, _D), lambda i: (0, 0)),
            pl.BlockSpec((1, 1, _BN), lambda i: (i, 0, 0)),
            pl.BlockSpec((_D, 64), lambda i: (0, 0)),
            pl.BlockSpec((1, 64), lambda i: (0, 0)),
            pl.BlockSpec((64, 32), lambda i: (0, 0)),
            pl.BlockSpec((1, 32), lambda i: (0, 0)),
            pl.BlockSpec((32, 1), lambda i: (0, 0)),
            pl.BlockSpec((1, 1), lambda i: (0, 0)),
        ],
        out_specs=pl.BlockSpec((_G, 1), lambda i: (0, 0)),
        out_shape=jax.ShapeDtypeStruct((_G, 1), jnp.float32),
        scratch_shapes=[
            pltpu.VMEM((_G, _D), jnp.float32),
            pltpu.VMEM((_G, _D), jnp.float32),
        ],
    )(accs, accs, xw2, dinv, b2, batch3, Wf1, bf1, Wf2, bf2, Wo, bo)


def kernel(x, edge_index, batch, W1, b1, W2, b2, Wf1, bf1, Wf2, bf2, Wo, bo):
    src = edge_index[0]
    dst = edge_index[1]
    src2 = src.reshape(32 * _CPT, _CH)
    dst2 = dst.reshape(32 * _CPT, _CH)

    degp = _deg_call(_deg_body)(dst)
    xw1, y1, dinv = _tc1(x, W1, degp.T)
    accs1 = _spmm_call(_spmm_body)(y1, src2, dst2)
    xw2, y2 = _tc2(accs1, xw1, dinv, W2, b1.reshape(1, _D))
    accs2 = _spmm_call(_spmm_body)(y2, src2, dst2)
    out = _tc3(accs2, xw2, dinv, b2.reshape(1, _D),
               batch.reshape(_N // _BN, 1, _BN),
               Wf1, bf1.reshape(1, 64), Wf2, bf2.reshape(1, 32),
               Wo, bo.reshape(1, 1))
    return out


# speedup vs baseline: 31.4189x; 1.1389x over previous
import jax
import jax.numpy as jnp
from jax import lax
from jax.experimental import pallas as pl
from jax.experimental.pallas import tpu as pltpu
from jax.experimental.pallas import tpu_sc as plsc

_N = 10000
_E = 320000
_D = 128
_G = 32
_CH = 80
_CPT = 125
_EPT = _E // 32
_RPS = _N // 16
_ZR = 25

def _mesh():
    return plsc.VectorSubcoreMesh(core_axis_name="c", subcore_axis_name="s",
                                  num_cores=2, num_subcores=16)


def _leaky(v):
    return jnp.where(v >= 0, v, 0.01 * v)


def _deg_body(dst_hbm, out_hbm, dstv, part):
    wid = lax.axis_index("c") * 16 + lax.axis_index("s")
    zero16 = jnp.zeros((16,), jnp.float32)
    ones16 = jnp.ones((16,), jnp.float32)

    def zrow(k, _):
        part[pl.ds(k * 16, 16)] = zero16
        return 0
    lax.fori_loop(0, _N // 16, zrow, 0)

    pltpu.sync_copy(dst_hbm.at[pl.ds(wid * _EPT, _EPT)], dstv)

    def step(k, _):
        idx = dstv[pl.ds(k * 16, 16)]
        plsc.addupdate_scatter(part, [idx], ones16)
        return 0
    lax.fori_loop(0, _EPT // 16, step, 0)

    pltpu.sync_copy(part, out_hbm.at[wid])


def _deg_call(f):
    return pl.kernel(
        f,
        mesh=_mesh(),
        compiler_params=pltpu.CompilerParams(needs_layout_passes=False),
        out_type=jax.ShapeDtypeStruct((32, _N), jnp.float32),
        scratch_types=[
            pltpu.VMEM((_EPT,), jnp.int32),
            pltpu.VMEM((_N,), jnp.float32),
        ],
    )


def _spmm_body(y_hbm, src_hbm, dst_hbm, out_hbm, srcv, dstv, rowsv0, rowsv1,
               rowsv2, acc, semz, semg0, semg1, semg2):
    cid = lax.axis_index("c")
    sid = lax.axis_index("s")
    wid = cid * 16 + sid
    zero16 = jnp.zeros((16,), jnp.float32)

    def zrow(r, _):
        for j in range(_D // 16):
            rowsv2[r, pl.ds(j * 16, 16)] = zero16
        return 0
    lax.fori_loop(0, _ZR, zrow, 0)

    def zslab(j, _):
        pltpu.async_copy(rowsv2.at[pl.ds(0, _ZR)],
                         acc.at[pl.ds(sid * _RPS + j * _ZR, _ZR)], semz)
        return 0
    lax.fori_loop(0, _RPS // _ZR, zslab, 0)

    pltpu.sync_copy(src_hbm.at[pl.ds(wid * _CPT, _CPT)], srcv)
    pltpu.sync_copy(dst_hbm.at[pl.ds(wid * _CPT, _CPT)], dstv)

    bufs = (rowsv0, rowsv1, rowsv2)
    sems = (semg0, semg1, semg2)
    pltpu.async_copy(y_hbm.at[srcv.at[0]], rowsv0, semg0)
    pltpu.async_copy(y_hbm.at[srcv.at[1]], rowsv1, semg1)

    def zdrain(j, _):
        pltpu.make_async_copy(rowsv2.at[pl.ds(0, _ZR)],
                              acc.at[pl.ds(sid * _RPS + j * _ZR, _ZR)],
                              semz).wait()
        return 0
    lax.fori_loop(0, _RPS // _ZR, zdrain, 0)
    plsc.subcore_barrier()


    def trip(j, _):
        k = j * 3
        for b in range(3):
            pltpu.make_async_copy(y_hbm.at[srcv.at[k + b]], bufs[b],
                                  sems[b]).wait()

            @pl.when(k + b + 2 < _CPT)
            def _():
                pltpu.async_copy(y_hbm.at[srcv.at[k + b + 2]],
                                 bufs[(b + 2) % 3], sems[(b + 2) % 3])

            pltpu.sync_copy(bufs[b], acc.at[dstv.at[k + b]], add=True)
        return 0
    lax.fori_loop(0, _CPT // 3, trip, 0)

    for k in range((_CPT // 3) * 3, _CPT):
        b = k % 3
        pltpu.make_async_copy(y_hbm.at[srcv.at[k]], bufs[b], sems[b]).wait()
        pltpu.sync_copy(bufs[b], acc.at[dstv.at[k]], add=True)

    plsc.subcore_barrier()
    pltpu.sync_copy(acc.at[pl.ds(sid * _RPS, _RPS)],
                    out_hbm.at[cid, pl.ds(sid * _RPS, _RPS)])


def _spmm_call(f):
    return pl.kernel(
        f,
        mesh=_mesh(),
        compiler_params=pltpu.CompilerParams(needs_layout_passes=False,
                                             use_tc_tiling_on_sc=False),
        out_type=jax.ShapeDtypeStruct((2, _N, _D), jnp.float32),
        scratch_types=[
            pltpu.VMEM((_CPT, _CH), jnp.int32),
            pltpu.VMEM((_CPT, _CH), jnp.int32),
            pltpu.VMEM((_CH, _D), jnp.float32),
            pltpu.VMEM((_CH, _D), jnp.float32),
            pltpu.VMEM((_CH, _D), jnp.float32),
            pltpu.VMEM_SHARED((_N, _D), jnp.float32),
            pltpu.SemaphoreType.DMA,
            pltpu.SemaphoreType.DMA,
            pltpu.SemaphoreType.DMA,
            pltpu.SemaphoreType.DMA,
        ],
    )


_BN = 1000


def _tc1_body(x_ref, w_ref, degp_ref, xw_ref, y_ref, dinv_ref):
    xw = jnp.dot(x_ref[...], w_ref[...], preferred_element_type=jnp.float32)
    deg = jnp.sum(degp_ref[...], axis=1) + 1.0
    dv = lax.rsqrt(deg)
    xw_ref[...] = xw
    y_ref[...] = xw * dv[:, None]
    dinv_ref[...] = dv[:, None]


def _tc1(x, W1, degp):
    return pl.pallas_call(
        _tc1_body,
        grid=(_N // _BN,),
        in_specs=[
            pl.BlockSpec((_BN, _D), lambda i: (i, 0)),
            pl.BlockSpec((_D, _D), lambda i: (0, 0)),
            pl.BlockSpec((_BN, 32), lambda i: (i, 0)),
        ],
        out_specs=[
            pl.BlockSpec((_BN, _D), lambda i: (i, 0)),
            pl.BlockSpec((_BN, _D), lambda i: (i, 0)),
            pl.BlockSpec((_BN, 1), lambda i: (i, 0)),
        ],
        out_shape=[
            jax.ShapeDtypeStruct((_N, _D), jnp.float32),
            jax.ShapeDtypeStruct((_N, _D), jnp.float32),
            jax.ShapeDtypeStruct((_N, 1), jnp.float32),
        ],
    )(x, W1, degp)


def _tc2_body(a0_ref, a1_ref, xw_ref, dinv_ref, w_ref, b_ref, xw2_ref, y2_ref):
    dv = dinv_ref[...]
    h = _leaky(dv * (a0_ref[0] + a1_ref[0]) + dv * dv * xw_ref[...] + b_ref[...])
    xw2 = jnp.dot(h, w_ref[...], preferred_element_type=jnp.float32)
    xw2_ref[...] = xw2
    y2_ref[...] = xw2 * dv


def _tc2(accs, xw1, dinv, W2, b1):
    return pl.pallas_call(
        _tc2_body,
        grid=(_N // _BN,),
        in_specs=[
            pl.BlockSpec((1, _BN, _D), lambda i: (0, i, 0)),
            pl.BlockSpec((1, _BN, _D), lambda i: (1, i, 0)),
            pl.BlockSpec((_BN, _D), lambda i: (i, 0)),
            pl.BlockSpec((_BN, 1), lambda i: (i, 0)),
            pl.BlockSpec((_D, _D), lambda i: (0, 0)),
            pl.BlockSpec((1, _D), lambda i: (0, 0)),
        ],
        out_specs=[
            pl.BlockSpec((_BN, _D), lambda i: (i, 0)),
            pl.BlockSpec((_BN, _D), lambda i: (i, 0)),
        ],
        out_shape=[
            jax.ShapeDtypeStruct((_N, _D), jnp.float32),
            jax.ShapeDtypeStruct((_N, _D), jnp.float32),
        ],
    )(accs, accs, xw1, dinv, W2, b1)


def _tc3_body(a0_ref, a1_ref, xw_ref, dinv_ref, b_ref, batch_ref,
              wf1_ref, bf1_ref, wf2_ref, bf2_ref, wo_ref, bo_ref,
              out_ref, pooled, cnt):
    i = pl.program_id(0)

    @pl.when(i == 0)
    def _():
        pooled[...] = jnp.zeros_like(pooled)
        cnt[...] = jnp.zeros_like(cnt)

    dv = dinv_ref[...]
    h = _leaky(dv * (a0_ref[0] + a1_ref[0]) + dv * dv * xw_ref[...] + b_ref[...])
    b = batch_ref[0, 0, :]
    gi = lax.broadcasted_iota(jnp.int32, (_G, _BN), 0)
    oh = (b[None, :] == gi).astype(jnp.float32)
    pooled[...] += jnp.dot(oh, h, preferred_element_type=jnp.float32,
                           precision=lax.Precision.HIGHEST)
    cnt[...] += jnp.broadcast_to(jnp.sum(oh, axis=1, keepdims=True), (_G, _D))

    @pl.when(i == _N // _BN - 1)
    def _():
        g = pooled[...] / jnp.maximum(cnt[...], 1.0)
        f = _leaky(jnp.dot(g, wf1_ref[...], preferred_element_type=jnp.float32)
                   + bf1_ref[...])
        f = _leaky(jnp.dot(f, wf2_ref[...], preferred_element_type=jnp.float32)
                   + bf2_ref[...])
        out_ref[...] = jnp.dot(f, wo_ref[...], preferred_element_type=jnp.float32) \
            + bo_ref[...]


def _tc3(accs, xw2, dinv, b2, batch3, Wf1, bf1, Wf2, bf2, Wo, bo):
    return pl.pallas_call(
        _tc3_body,
        grid=(_N // _BN,),
        in_specs=[
            pl.BlockSpec((1, _BN, _D), lambda i: (0, i, 0)),
            pl.BlockSpec((1, _BN, _D), lambda i: (1, i, 0)),
            pl.BlockSpec((_BN, _D), lambda i: (i, 0)),
            pl.BlockSpec((_BN, 1), lambda i: (i, 0)),
            pl.BlockSpec((1, _D), lambda i: (0, 0)),
            pl.BlockSpec((1, 1, _BN), lambda i: (i, 0, 0)),
            pl.BlockSpec((_D, 64), lambda i: (0, 0)),
            pl.BlockSpec((1, 64), lambda i: (0, 0)),
            pl.BlockSpec((64, 32), lambda i: (0, 0)),
            pl.BlockSpec((1, 32), lambda i: (0, 0)),
            pl.BlockSpec((32, 1), lambda i: (0, 0)),
            pl.BlockSpec((1, 1), lambda i: (0, 0)),
        ],
        out_specs=pl.BlockSpec((_G, 1), lambda i: (0, 0)),
        out_shape=jax.ShapeDtypeStruct((_G, 1), jnp.float32),
        scratch_shapes=[
            pltpu.VMEM((_G, _D), jnp.float32),
            pltpu.VMEM((_G, _D), jnp.float32),
        ],
    )(accs, accs, xw2, dinv, b2, batch3, Wf1, bf1, Wf2, bf2, Wo, bo)


def kernel(x, edge_index, batch, W1, b1, W2, b2, Wf1, bf1, Wf2, bf2, Wo, bo):
    src = edge_index[0]
    dst = edge_index[1]
    src2 = src.reshape(32 * _CPT, _CH)
    dst2 = dst.reshape(32 * _CPT, _CH)

    degp = _deg_call(_deg_body)(dst)
    xw1, y1, dinv = _tc1(x, W1, degp.T)
    accs1 = _spmm_call(_spmm_body)(y1, src2, dst2)
    xw2, y2 = _tc2(accs1, xw1, dinv, W2, b1.reshape(1, _D))
    accs2 = _spmm_call(_spmm_body)(y2, src2, dst2)
    out = _tc3(accs2, xw2, dinv, b2.reshape(1, _D),
               batch.reshape(_N // _BN, 1, _BN),
               Wf1, bf1.reshape(1, 64), Wf2, bf2.reshape(1, 32),
               Wo, bo.reshape(1, 1))
    return out


# speedup vs baseline: 32.1076x; 1.0219x over previous
import jax
import jax.numpy as jnp
from jax import lax
from jax.experimental import pallas as pl
from jax.experimental.pallas import tpu as pltpu
from jax.experimental.pallas import tpu_sc as plsc

_N = 10000
_E = 320000
_D = 128
_G = 32
_CH = 80
_CPT = 125
_EPT = _E // 32
_RPS = _N // 16
_ZR = 25

def _mesh():
    return plsc.VectorSubcoreMesh(core_axis_name="c", subcore_axis_name="s",
                                  num_cores=2, num_subcores=16)


def _leaky(v):
    return jnp.where(v >= 0, v, 0.01 * v)


def _deg_body(dst_hbm, out_hbm, dstv, part):
    wid = lax.axis_index("c") * 16 + lax.axis_index("s")
    zero16 = jnp.zeros((16,), jnp.float32)
    ones16 = jnp.ones((16,), jnp.float32)

    def zrow(k, _):
        part[pl.ds(k * 16, 16)] = zero16
        return 0
    lax.fori_loop(0, _N // 16, zrow, 0)

    pltpu.sync_copy(dst_hbm.at[pl.ds(wid * _EPT, _EPT)], dstv)

    def step(k, _):
        idx = dstv[pl.ds(k * 16, 16)]
        plsc.addupdate_scatter(part, [idx], ones16)
        return 0
    lax.fori_loop(0, _EPT // 16, step, 0)

    pltpu.sync_copy(part, out_hbm.at[wid])


def _deg_call(f):
    return pl.kernel(
        f,
        mesh=_mesh(),
        compiler_params=pltpu.CompilerParams(needs_layout_passes=False),
        out_type=jax.ShapeDtypeStruct((32, _N), jnp.float32),
        scratch_types=[
            pltpu.VMEM((_EPT,), jnp.int32),
            pltpu.VMEM((_N,), jnp.float32),
        ],
    )


def _spmm_body(y_hbm, src_hbm, dst_hbm, out_hbm, srcv, dstv, rowsv0, rowsv1,
               rowsv2, acc, semz, semg0, semg1, semg2):
    cid = lax.axis_index("c")
    sid = lax.axis_index("s")
    wid = cid * 16 + sid
    zero16 = jnp.zeros((16,), jnp.float32)

    def zrow(r, _):
        for j in range(_D // 16):
            rowsv2[r, pl.ds(j * 16, 16)] = zero16
        return 0
    lax.fori_loop(0, _ZR, zrow, 0)

    def zslab(j, _):
        pltpu.async_copy(rowsv2.at[pl.ds(0, _ZR)],
                         acc.at[pl.ds(sid * _RPS + j * _ZR, _ZR)], semz)
        return 0
    lax.fori_loop(0, _RPS // _ZR, zslab, 0)

    pltpu.sync_copy(src_hbm.at[pl.ds(wid * _CPT, _CPT)], srcv)
    pltpu.sync_copy(dst_hbm.at[pl.ds(wid * _CPT, _CPT)], dstv)

    bufs = (rowsv0, rowsv1, rowsv2)
    sems = (semg0, semg1, semg2)
    pltpu.async_copy(y_hbm.at[srcv.at[0]], rowsv0, semg0)
    pltpu.async_copy(y_hbm.at[srcv.at[1]], rowsv1, semg1)

    def zdrain(j, _):
        pltpu.make_async_copy(rowsv2.at[pl.ds(0, _ZR)],
                              acc.at[pl.ds(sid * _RPS + j * _ZR, _ZR)],
                              semz).wait()
        return 0
    lax.fori_loop(0, _RPS // _ZR, zdrain, 0)
    plsc.subcore_barrier()


    def trip(j, _):
        k = j * 3
        for b in range(3):
            pltpu.make_async_copy(y_hbm.at[srcv.at[k + b]], bufs[b],
                                  sems[b]).wait()

            @pl.when(k + b + 2 < _CPT)
            def _():
                pltpu.async_copy(y_hbm.at[srcv.at[k + b + 2]],
                                 bufs[(b + 2) % 3], sems[(b + 2) % 3])

            pltpu.sync_copy(bufs[b], acc.at[dstv.at[k + b]], add=True)
        return 0
    lax.fori_loop(0, _CPT // 3, trip, 0)

    for k in range((_CPT // 3) * 3, _CPT):
        b = k % 3
        pltpu.make_async_copy(y_hbm.at[srcv.at[k]], bufs[b], sems[b]).wait()
        pltpu.sync_copy(bufs[b], acc.at[dstv.at[k]], add=True)

    plsc.subcore_barrier()
    pltpu.sync_copy(acc.at[pl.ds(sid * _RPS, _RPS)],
                    out_hbm.at[cid, pl.ds(sid * _RPS, _RPS)])


def _spmm_call(f):
    return pl.kernel(
        f,
        mesh=_mesh(),
        compiler_params=pltpu.CompilerParams(needs_layout_passes=False,
                                             use_tc_tiling_on_sc=False),
        out_type=jax.ShapeDtypeStruct((2, _N, _D), jnp.float32),
        scratch_types=[
            pltpu.VMEM((_CPT, _CH), jnp.int32),
            pltpu.VMEM((_CPT, _CH), jnp.int32),
            pltpu.VMEM((_CH, _D), jnp.float32),
            pltpu.VMEM((_CH, _D), jnp.float32),
            pltpu.VMEM((_CH, _D), jnp.float32),
            pltpu.VMEM_SHARED((_N, _D), jnp.float32),
            pltpu.SemaphoreType.DMA,
            pltpu.SemaphoreType.DMA,
            pltpu.SemaphoreType.DMA,
            pltpu.SemaphoreType.DMA,
        ],
    )


_BN = 2000


def _tc0_body(x_ref, w_ref, xw_ref):
    xw_ref[...] = jnp.dot(x_ref[...], w_ref[...],
                          preferred_element_type=jnp.float32)


def _tc0(x, W1):
    return pl.pallas_call(
        _tc0_body,
        grid=(_N // _BN,),
        in_specs=[
            pl.BlockSpec((_BN, _D), lambda i: (i, 0)),
            pl.BlockSpec((_D, _D), lambda i: (0, 0)),
        ],
        out_specs=pl.BlockSpec((_BN, _D), lambda i: (i, 0)),
        out_shape=jax.ShapeDtypeStruct((_N, _D), jnp.float32),
    )(x, W1)


def _tc1_body(xw_ref, degp_ref, y_ref, dinv_ref):
    deg = jnp.sum(degp_ref[...], axis=1) + 1.0
    dv = lax.rsqrt(deg)
    y_ref[...] = xw_ref[...] * dv[:, None]
    dinv_ref[...] = dv[:, None]


def _tc1(xw, degp):
    return pl.pallas_call(
        _tc1_body,
        grid=(_N // _BN,),
        in_specs=[
            pl.BlockSpec((_BN, _D), lambda i: (i, 0)),
            pl.BlockSpec((_BN, 32), lambda i: (i, 0)),
        ],
        out_specs=[
            pl.BlockSpec((_BN, _D), lambda i: (i, 0)),
            pl.BlockSpec((_BN, 1), lambda i: (i, 0)),
        ],
        out_shape=[
            jax.ShapeDtypeStruct((_N, _D), jnp.float32),
            jax.ShapeDtypeStruct((_N, 1), jnp.float32),
        ],
    )(xw, degp)


def _tc2_body(a0_ref, a1_ref, xw_ref, dinv_ref, w_ref, b_ref, xw2_ref, y2_ref):
    dv = dinv_ref[...]
    h = _leaky(dv * (a0_ref[0] + a1_ref[0]) + dv * dv * xw_ref[...] + b_ref[...])
    xw2 = jnp.dot(h, w_ref[...], preferred_element_type=jnp.float32)
    xw2_ref[...] = xw2
    y2_ref[...] = xw2 * dv


def _tc2(accs, xw1, dinv, W2, b1):
    return pl.pallas_call(
        _tc2_body,
        grid=(_N // _BN,),
        in_specs=[
            pl.BlockSpec((1, _BN, _D), lambda i: (0, i, 0)),
            pl.BlockSpec((1, _BN, _D), lambda i: (1, i, 0)),
            pl.BlockSpec((_BN, _D), lambda i: (i, 0)),
            pl.BlockSpec((_BN, 1), lambda i: (i, 0)),
            pl.BlockSpec((_D, _D), lambda i: (0, 0)),
            pl.BlockSpec((1, _D), lambda i: (0, 0)),
        ],
        out_specs=[
            pl.BlockSpec((_BN, _D), lambda i: (i, 0)),
            pl.BlockSpec((_BN, _D), lambda i: (i, 0)),
        ],
        out_shape=[
            jax.ShapeDtypeStruct((_N, _D), jnp.float32),
            jax.ShapeDtypeStruct((_N, _D), jnp.float32),
        ],
    )(accs, accs, xw1, dinv, W2, b1)


def _tc3_body(a0_ref, a1_ref, xw_ref, dinv_ref, b_ref, batch_ref,
              wf1_ref, bf1_ref, wf2_ref, bf2_ref, wo_ref, bo_ref,
              out_ref, pooled, cnt):
    i = pl.program_id(0)

    @pl.when(i == 0)
    def _():
        pooled[...] = jnp.zeros_like(pooled)
        cnt[...] = jnp.zeros_like(cnt)

    dv = dinv_ref[...]
    h = _leaky(dv * (a0_ref[0] + a1_ref[0]) + dv * dv * xw_ref[...] + b_ref[...])
    b = batch_ref[0, 0, :]
    gi = lax.broadcasted_iota(jnp.int32, (_G, _BN), 0)
    oh = (b[None, :] == gi).astype(jnp.float32)
    pooled[...] += jnp.dot(oh, h, preferred_element_type=jnp.float32,
                           precision=lax.Precision.HIGHEST)
    cnt[...] += jnp.broadcast_to(jnp.sum(oh, axis=1, keepdims=True), (_G, _D))

    @pl.when(i == _N // _BN - 1)
    def _():
        g = pooled[...] / jnp.maximum(cnt[...], 1.0)
        f = _leaky(jnp.dot(g, wf1_ref[...], preferred_element_type=jnp.float32)
                   + bf1_ref[...])
        f = _leaky(jnp.dot(f, wf2_ref[...], preferred_element_type=jnp.float32)
                   + bf2_ref[...])
        out_ref[...] = jnp.dot(f, wo_ref[...], preferred_element_type=jnp.float32) \
            + bo_ref[...]


def _tc3(accs, xw2, dinv, b2, batch3, Wf1, bf1, Wf2, bf2, Wo, bo):
    return pl.pallas_call(
        _tc3_body,
        grid=(_N // _BN,),
        in_specs=[
            pl.BlockSpec((1, _BN, _D), lambda i: (0, i, 0)),
            pl.BlockSpec((1, _BN, _D), lambda i: (1, i, 0)),
            pl.BlockSpec((_BN, _D), lambda i: (i, 0)),
            pl.BlockSpec((_BN, 1), lambda i: (i, 0)),
            pl.BlockSpec((1, _D), lambda i: (0, 0)),
            pl.BlockSpec((1, 1, _BN), lambda i: (i, 0, 0)),
            pl.BlockSpec((_D, 64), lambda i: (0, 0)),
            pl.BlockSpec((1, 64), lambda i: (0, 0)),
            pl.BlockSpec((64, 32), lambda i: (0, 0)),
            pl.BlockSpec((1, 32), lambda i: (0, 0)),
            pl.BlockSpec((32, 1), lambda i: (0, 0)),
            pl.BlockSpec((1, 1), lambda i: (0, 0)),
        ],
        out_specs=pl.BlockSpec((_G, 1), lambda i: (0, 0)),
        out_shape=jax.ShapeDtypeStruct((_G, 1), jnp.float32),
        scratch_shapes=[
            pltpu.VMEM((_G, _D), jnp.float32),
            pltpu.VMEM((_G, _D), jnp.float32),
        ],
    )(accs, accs, xw2, dinv, b2, batch3, Wf1, bf1, Wf2, bf2, Wo, bo)


def kernel(x, edge_index, batch, W1, b1, W2, b2, Wf1, bf1, Wf2, bf2, Wo, bo):
    src = edge_index[0]
    dst = edge_index[1]
    src2 = src.reshape(32 * _CPT, _CH)
    dst2 = dst.reshape(32 * _CPT, _CH)

    degp = _deg_call(_deg_body)(dst)
    xw1 = _tc0(x, W1)
    y1, dinv = _tc1(xw1, degp.T)
    accs1 = _spmm_call(_spmm_body)(y1, src2, dst2)
    xw2, y2 = _tc2(accs1, xw1, dinv, W2, b1.reshape(1, _D))
    accs2 = _spmm_call(_spmm_body)(y2, src2, dst2)
    out = _tc3(accs2, xw2, dinv, b2.reshape(1, _D),
               batch.reshape(_N // _BN, 1, _BN),
               Wf1, bf1.reshape(1, 64), Wf2, bf2.reshape(1, 32),
               Wo, bo.reshape(1, 1))
    return out
